# baseline (speedup 1.0000x reference)
import jax
import jax.numpy as jnp
from jax.experimental import pallas as pl
from jax.experimental.pallas import tpu as pltpu

_INPUT_DIM = 3
_D = 4
_L = 5
_NEG = 0.01
_BASE = _INPUT_DIM + 1
_STRIDE = 2 * _D + 1
_ROWS = _BASE + _L * _STRIDE
_COLS = 4 * _D
_NSPLIT = 8
_M_SCALE = 64.0
_INV_SCALE = 1.0 / _M_SCALE
_KT = 256


def _proj(wT, htp):
    acc = wT[:, 0:1] * htp[0:1, :]
    for d in range(1, wT.shape[1]):
        acc = acc + wT[:, d:d + 1] * htp[d:d + 1, :]
    return acc


def _enc_kernel(x_ref, *rest):
    a_refs = rest[:_NSPLIT]
    p_ref = rest[_NSPLIT]
    o_ref = rest[_NSPLIT + 1]
    D = _D
    x = x_ref[...]
    p = p_ref[...]
    a8 = [r[...].astype(jnp.float8_e5m2) for r in a_refs]

    N = x.shape[0]
    S = N // _NSPLIT
    nk = N // _KT

    wTs, bTs = [], []
    for l in range(_L):
        r0 = _BASE + l * _STRIDE
        wfT = jnp.swapaxes(p[r0:r0 + 2 * D, :], 0, 1)
        wTs.append(jnp.concatenate([wfT[:, 0:D] * _M_SCALE,
                                    wfT[:, D:2 * D]], axis=0))
        bTs.append(jnp.swapaxes(p[r0 + 2 * D:r0 + 2 * D + 1, :], 0, 1))

    w1T = jnp.swapaxes(p[0:_INPUT_DIM, 0:D], 0, 1)
    b1T = jnp.swapaxes(p[_INPUT_DIM:_INPUT_DIM + 1, 0:D], 0, 1)
    xT = jnp.swapaxes(x, 0, 1)
    ht0 = _proj(w1T, xT) + b1T
    ht0 = jnp.where(ht0 >= 0, ht0, _NEG * ht0)

    ht_parts = [ht0[:, s * S:(s + 1) * S] for s in range(_NSPLIT)]
    m8_parts = [None] * _NSPLIT
    ghT_parts = [None] * _NSPLIT
    for s in range(_NSPLIT):
        c = _proj(wTs[0], ht_parts[s])
        m8_parts[s] = jnp.swapaxes(jnp.clip(c[0:4 * D], -448.0, 448.0),
                                   0, 1).astype(jnp.float8_e4m3fn)
        ghT_parts[s] = c[4 * D:8 * D] + bTs[0]

    mxu_of = [s % 2 for s in range(_NSPLIT)]
    addr_of = [(s // 2) * (_KT // 4) for s in range(_NSPLIT)]

    mxT_parts = [None] * _NSPLIT
    for l in range(_L):
        m8 = jnp.concatenate(m8_parts, axis=0)
        m256 = jnp.concatenate(
            [m8, jnp.zeros((N, _KT - _COLS), m8.dtype)], axis=1)

        for k in range(nk):
            tile = m256[k * _KT:(k + 1) * _KT, :]
            pltpu.matmul_push_rhs(tile, staging_register=k % 2, mxu_index=0)
            pltpu.matmul_push_rhs(tile, staging_register=k % 2, mxu_index=1)
            for s in range(_NSPLIT):
                pltpu.matmul_acc_lhs(
                    addr_of[s], a8[s][:, k * _KT:(k + 1) * _KT],
                    mxu_index=mxu_of[s],
                    load_staged_rhs=(k % 2) if s < 2 else None)

        last = l == _L - 1
        for s in range(_NSPLIT):
            gi = pltpu.matmul_pop(addr_of[s], (S, _KT), jnp.float32,
                                  mxu_index=mxu_of[s])[:, 0:_COLS]
            gt = jnp.swapaxes(gi, 0, 1) * _INV_SCALE + ghT_parts[s]
            r = jax.nn.sigmoid(gt[0 * D:1 * D])
            z = jax.nn.sigmoid(gt[1 * D:2 * D])
            n = jnp.tanh(gt[2 * D:3 * D] + (r - 1.0) * gt[3 * D:4 * D])
            hs = n + z * (ht_parts[s] - n)
            ht_parts[s] = hs
            mxT_parts[s] = hs if l == 0 else jnp.maximum(mxT_parts[s], hs)
            if not last:
                c = _proj(wTs[l + 1], hs)
                m8_parts[s] = jnp.swapaxes(jnp.clip(c[0:4 * D], -448.0, 448.0),
                                           0, 1).astype(jnp.float8_e4m3fn)
                ghT_parts[s] = c[4 * D:8 * D] + bTs[l + 1]

    for s in range(_NSPLIT):
        o_ref[s * S:(s + 1) * S, :] = jnp.swapaxes(mxT_parts[s], 0, 1)


def kernel(x, a_hat, packed_params):
    B, N, _ = x.shape
    S = N // _NSPLIT
    a_specs = [
        pl.BlockSpec((None, S, N), lambda b, i=i: (b, i, 0))
        for i in range(_NSPLIT)
    ]
    return pl.pallas_call(
        _enc_kernel,
        out_shape=jax.ShapeDtypeStruct((B, N, _D), jnp.float32),
        grid_spec=pltpu.PrefetchScalarGridSpec(
            num_scalar_prefetch=0,
            grid=(B,),
            in_specs=[
                pl.BlockSpec((None, N, _INPUT_DIM), lambda b: (b, 0, 0)),
                *a_specs,
                pl.BlockSpec((_ROWS, _COLS), lambda b: (0, 0)),
            ],
            out_specs=pl.BlockSpec((None, N, _D), lambda b: (b, 0, 0)),
        ),
        compiler_params=pltpu.CompilerParams(
            dimension_semantics=("arbitrary",),
        ),
    )(x, *([a_hat] * _NSPLIT), packed_params)


# speedup vs baseline: 1.1268x; 1.1268x over previous
import jax
import jax.numpy as jnp
from jax.experimental import pallas as pl
from jax.experimental.pallas import tpu as pltpu

_INPUT_DIM = 3
_D = 4
_L = 5
_NEG = 0.01
_BASE = _INPUT_DIM + 1
_STRIDE = 2 * _D + 1
_ROWS = _BASE + _L * _STRIDE
_COLS = 4 * _D
_NSPLIT = 4
_M_SCALE = 64.0
_INV_SCALE = 1.0 / _M_SCALE


def _proj(wT, htp):
    acc = wT[:, 0:1] * htp[0:1, :]
    for d in range(1, _D):
        acc = acc + wT[:, d:d + 1] * htp[d:d + 1, :]
    return acc


def _enc_kernel(x_ref, *rest):
    a_refs = rest[:_NSPLIT]
    p_ref = rest[_NSPLIT]
    o_ref = rest[_NSPLIT + 1]
    D = _D
    x = x_ref[...]
    p = p_ref[...]
    a8 = [r[...].astype(jnp.float8_e5m2) for r in a_refs]

    N = x.shape[0]
    S = N // _NSPLIT

    wTs, bTs = [], []
    for l in range(_L):
        r0 = _BASE + l * _STRIDE
        wfT = jnp.swapaxes(p[r0:r0 + 2 * D, :], 0, 1)
        wTs.append(jnp.concatenate([wfT[:, 0:D] * _M_SCALE,
                                    wfT[:, D:2 * D]], axis=0))
        bTs.append(jnp.swapaxes(p[r0 + 2 * D:r0 + 2 * D + 1, :], 0, 1))

    w_first = p[0:_INPUT_DIM, 0:D]
    b_first = p[_INPUT_DIM:_INPUT_DIM + 1, 0:D]
    h0 = jnp.dot(x, w_first, preferred_element_type=jnp.float32) + b_first
    h0 = jnp.where(h0 >= 0, h0, _NEG * h0)
    ht0 = jnp.swapaxes(h0, 0, 1)

    ht_parts = [ht0[:, s * S:(s + 1) * S] for s in range(_NSPLIT)]
    m16_parts = [None] * _NSPLIT
    ghT_parts = [None] * _NSPLIT
    for s in range(_NSPLIT):
        c = _proj(wTs[0], ht_parts[s])
        m16_parts[s] = jnp.swapaxes(jnp.clip(c[0:4 * D], -448.0, 448.0),
                                    0, 1).astype(jnp.float8_e4m3fn)
        ghT_parts[s] = c[4 * D:8 * D] + bTs[0]

    mxT_parts = [None] * _NSPLIT
    for l in range(_L):
        m16 = jnp.concatenate(m16_parts, axis=0)
        last = l == _L - 1
        for s in range(_NSPLIT):
            gi = jnp.dot(a8[s], m16, preferred_element_type=jnp.float32)
            gt = jnp.swapaxes(gi, 0, 1) * _INV_SCALE + ghT_parts[s]
            r = jax.nn.sigmoid(gt[0 * D:1 * D])
            z = jax.nn.sigmoid(gt[1 * D:2 * D])
            n = jnp.tanh(gt[2 * D:3 * D] + (r - 1.0) * gt[3 * D:4 * D])
            hs = n + z * (ht_parts[s] - n)
            ht_parts[s] = hs
            mxT_parts[s] = hs if l == 0 else jnp.maximum(mxT_parts[s], hs)
            if not last:
                c = _proj(wTs[l + 1], hs)
                m16_parts[s] = jnp.swapaxes(jnp.clip(c[0:4 * D], -448.0, 448.0),
                                            0, 1).astype(jnp.float8_e4m3fn)
                ghT_parts[s] = c[4 * D:8 * D] + bTs[l + 1]

    for s in range(_NSPLIT):
        o_ref[s * S:(s + 1) * S, :] = jnp.swapaxes(mxT_parts[s], 0, 1)


def kernel(x, a_hat, packed_params):
    B, N, _ = x.shape
    S = N // _NSPLIT
    a_specs = [
        pl.BlockSpec((None, S, N), lambda b, i=i: (b, i, 0))
        for i in range(_NSPLIT)
    ]
    return pl.pallas_call(
        _enc_kernel,
        out_shape=jax.ShapeDtypeStruct((B, N, _D), jnp.float32),
        grid_spec=pltpu.PrefetchScalarGridSpec(
            num_scalar_prefetch=0,
            grid=(B,),
            in_specs=[
                pl.BlockSpec((None, N, _INPUT_DIM), lambda b: (b, 0, 0)),
                *a_specs,
                pl.BlockSpec((_ROWS, _COLS), lambda b: (0, 0)),
            ],
            out_specs=pl.BlockSpec((None, N, _D), lambda b: (b, 0, 0)),
        ),
        compiler_params=pltpu.CompilerParams(
            dimension_semantics=("arbitrary",),
        ),
    )(x, *([a_hat] * _NSPLIT), packed_params)
